# baseline (speedup 1.0000x reference)
import functools

import jax
import jax.numpy as jnp
from jax import lax
from jax.experimental import pallas as pl
from jax.experimental.pallas import tpu as pltpu
from jax.experimental.pallas import tpu_sc as plsc

NC = 2
NS = 16
NW = NC * NS
K = 128


CHR = 80


@functools.lru_cache(maxsize=None)
def _deg_kernel(N, NCH):
    NCHR = N // CHR
    MAXC = (NCHR + NS - 1) // NS
    mesh = plsc.VectorSubcoreMesh(core_axis_name="c", subcore_axis_name="s")

    @functools.partial(
        pl.kernel,
        out_type=jax.ShapeDtypeStruct((NC, N, 16), jnp.float32),
        mesh=mesh,
        scratch_types=[
            pltpu.VMEM_SHARED((N + 64, 16), jnp.float32),
            pltpu.VMEM((K,), jnp.int32),
            pltpu.VMEM((K,), jnp.int32),
            pltpu.VMEM((K, 16), jnp.float32),
            pltpu.VMEM((CHR, 16), jnp.float32),
            pltpu.SemaphoreType.DMA,
            pltpu.SemaphoreType.DMA,
        ],
    )
    def deg(ei_hbm, out_hbm, acc, idx0, idx1, ones_v, zbuf, isem0, isem1):
        c = lax.axis_index("c")
        s = lax.axis_index("s")
        wid = s * NC + c

        def fill(i, _):
            ones_v[i] = jnp.ones((16,), jnp.float32)
            zbuf[i % CHR] = jnp.zeros((16,), jnp.float32)
            return 0
        lax.fori_loop(0, K, fill, 0)

        def zchunk(k, _):
            idx = s + k * NS

            @pl.when(idx < NCHR)
            def _():
                pltpu.sync_copy(zbuf, acc.at[pl.ds(idx * CHR, CHR)])
            return 0
        lax.fori_loop(0, MAXC, zchunk, 0)
        plsc.subcore_barrier()

        def chunk(i, _):
            pltpu.sync_copy(ei_hbm.at[wid, 2 * i + 1], idx0)
            pltpu.sync_copy(ones_v, acc.at[idx0], add=True)
            return 0
        lax.fori_loop(0, NCH, chunk, 0)
        plsc.subcore_barrier()

        def rchunk(k, _):
            idx = s + k * NS

            @pl.when(idx < NCHR)
            def _():
                r0 = idx * CHR
                pltpu.sync_copy(acc.at[pl.ds(r0, CHR)], zbuf)
                pltpu.sync_copy(zbuf, out_hbm.at[c, pl.ds(r0, CHR)])
            return 0
        lax.fori_loop(0, MAXC, rchunk, 0)

    return deg


@functools.lru_cache(maxsize=None)
def _agg_kernel(N, NCH, D):
    NCHR = N // CHR
    MAXC = (NCHR + NS - 1) // NS
    mesh = plsc.VectorSubcoreMesh(core_axis_name="c", subcore_axis_name="s")

    @functools.partial(
        pl.kernel,
        out_type=jax.ShapeDtypeStruct((NC, N, D), jnp.float32),
        mesh=mesh,
        scratch_types=[
            pltpu.VMEM_SHARED((N + 64, D), jnp.float32),
            pltpu.VMEM((K,), jnp.int32),
            pltpu.VMEM((K,), jnp.int32),
            pltpu.VMEM((K,), jnp.int32),
            pltpu.VMEM((K,), jnp.int32),
            pltpu.VMEM((K, D), jnp.float32),
            pltpu.VMEM((K, D), jnp.float32),
            pltpu.SemaphoreType.DMA,
            pltpu.SemaphoreType.DMA,
            pltpu.SemaphoreType.DMA,
            pltpu.SemaphoreType.DMA,
        ],
    )
    def agg(z_hbm, ei_hbm, out_hbm, acc, srci0, srci1, dsti0, dsti1,
            rows0, rows1, isem0, isem1, rsem0, rsem1):
        c = lax.axis_index("c")
        s = lax.axis_index("s")
        wid = s * NC + c

        srcb = (srci0, srci1)
        dstb = (dsti0, dsti1)
        rowsb = (rows0, rows1)
        isems = (isem0, isem1)
        rsems = (rsem0, rsem1)

        pltpu.sync_copy(ei_hbm.at[wid, 0, 0], srci0)
        pltpu.sync_copy(ei_hbm.at[wid, 0, 1], dsti0)
        pltpu.async_copy(z_hbm.at[srci0], rows0, rsem0)
        pltpu.async_copy(ei_hbm.at[wid, 1, 0], srci1, isem1)
        pltpu.async_copy(ei_hbm.at[wid, 1, 1], dsti1, isem1)

        zbuf = rows1.at[pl.ds(0, CHR)]

        def fill(i, _):
            for j in range(D // 16):
                rows1[i, pl.ds(j * 16, 16)] = jnp.zeros((16,), jnp.float32)
            return 0
        lax.fori_loop(0, CHR, fill, 0)

        def zchunk(k, _):
            idx = s + k * NS

            @pl.when(idx < NCHR)
            def _():
                pltpu.async_copy(zbuf, acc.at[pl.ds(idx * CHR, CHR)],
                                 isem0)
            return 0
        lax.fori_loop(0, MAXC, zchunk, 0)

        def zwait(k, _):
            idx = s + k * NS

            @pl.when(idx < NCHR)
            def _():
                pltpu.make_async_copy(
                    zbuf, acc.at[pl.ds(idx * CHR, CHR)], isem0).wait()
            return 0
        lax.fori_loop(0, MAXC, zwait, 0)
        plsc.subcore_barrier()

        def step(cur, b):
            nb = 1 - b
            nxt = cur + 1

            @pl.when(nxt < NCH)
            def _():
                pltpu.make_async_copy(ei_hbm.at[wid, nxt, 0], srcb[nb],
                                      isems[nb]).wait()
                pltpu.make_async_copy(ei_hbm.at[wid, nxt, 1], dstb[nb],
                                      isems[nb]).wait()
                pltpu.async_copy(z_hbm.at[srcb[nb]], rowsb[nb], rsems[nb])
            pltpu.make_async_copy(z_hbm.at[srcb[b]], rowsb[b],
                                  rsems[b]).wait()
            pltpu.sync_copy(rowsb[b], acc.at[dstb[b]], add=True)
            nxt2 = cur + 2

            @pl.when(nxt2 < NCH)
            def _():
                pltpu.async_copy(ei_hbm.at[wid, nxt2, 0], srcb[b], isems[b])
                pltpu.async_copy(ei_hbm.at[wid, nxt2, 1], dstb[b], isems[b])

        def pair(i, _):
            for b in range(2):
                cur = 2 * i + b

                @pl.when(cur < NCH)
                def _():
                    step(cur, b)
            return 0
        lax.fori_loop(0, (NCH + 1) // 2, pair, 0)
        plsc.subcore_barrier()

        rb = (rows0.at[pl.ds(0, CHR)], rows1.at[pl.ds(0, CHR)])

        def rpair(i, _):
            for b in range(2):
                k = 2 * i + b
                idx = s + k * NS

                @pl.when(idx < NCHR)
                def _():
                    @pl.when(k >= 2)
                    def _():
                        prv = (s + (k - 2) * NS) * CHR
                        pltpu.make_async_copy(
                            rb[b], out_hbm.at[c, pl.ds(prv, CHR)],
                            rsems[b]).wait()
                    r0 = idx * CHR
                    pltpu.sync_copy(acc.at[pl.ds(r0, CHR)], rb[b])
                    pltpu.async_copy(rb[b], out_hbm.at[c, pl.ds(r0, CHR)],
                                     rsems[b])
            return 0
        lax.fori_loop(0, (MAXC + 1) // 2, rpair, 0)
        for b in range(2):
            pltpu.make_async_copy(rb[b], out_hbm.at[c, pl.ds(0, CHR)],
                                  rsems[b]).wait()

    return agg


def _tc_mm0(x, w0):
    N, D = x.shape

    def body(x_ref, w_ref, u_ref):
        u_ref[...] = jnp.dot(x_ref[...], w_ref[...],
                             preferred_element_type=jnp.float32)

    return pl.pallas_call(
        body,
        out_shape=jax.ShapeDtypeStruct((N, D), jnp.float32),
    )(x, w0)


def _tc_init(u, degp):
    N, D = u.shape

    def body(u_ref, degp_ref, dinv_ref, z_ref):
        deg = degp_ref[0, :, 0:1] + degp_ref[1, :, 0:1] + 1.0
        dinv = lax.rsqrt(deg)
        dinv_ref[...] = dinv
        z_ref[...] = u_ref[...] * dinv

    return pl.pallas_call(
        body,
        out_shape=[jax.ShapeDtypeStruct((N, 1), jnp.float32),
                   jax.ShapeDtypeStruct((N, D), jnp.float32)],
    )(u, degp)


def _tc_layer(S, z, dinv, h_prev, b, g, be, w_next, with_res):
    N, D = z.shape

    def body(*refs):
        if with_res:
            (s_ref, z_ref, dinv_ref, h_ref, b_ref, g_ref, be_ref, w_ref,
             hn_ref, zn_ref) = refs
        else:
            (s_ref, z_ref, dinv_ref, b_ref, g_ref, be_ref, w_ref,
             hn_ref, zn_ref) = refs
        dinv = dinv_ref[...]
        agg = dinv * (s_ref[0] + s_ref[1] + z_ref[...]) + b_ref[...]
        mu = jnp.mean(agg, axis=0, keepdims=True)
        var = jnp.mean((agg - mu) ** 2, axis=0, keepdims=True)
        h = (agg - mu) * lax.rsqrt(var + 1e-5) * g_ref[...] + be_ref[...]
        h = jnp.maximum(h, 0.0)
        if with_res:
            h = h + h_ref[...]
        hn_ref[...] = h
        zn_ref[...] = jnp.dot(h, w_ref[...],
                              preferred_element_type=jnp.float32) * dinv

    args = [S, z, dinv] + ([h_prev] if with_res else []) + \
        [b.reshape(1, D), g.reshape(1, D), be.reshape(1, D), w_next]
    return pl.pallas_call(
        body,
        out_shape=[jax.ShapeDtypeStruct((N, D), jnp.float32),
                   jax.ShapeDtypeStruct((N, D), jnp.float32)],
    )(*args)


def _tc_final(S, z, dinv, b):
    N, D = z.shape

    def body(s_ref, z_ref, dinv_ref, b_ref, o_ref):
        o_ref[...] = dinv_ref[...] * (s_ref[0] + s_ref[1] + z_ref[...]) \
            + b_ref[...]

    return pl.pallas_call(
        body,
        out_shape=jax.ShapeDtypeStruct((N, D), jnp.float32),
    )(S, z, dinv, b.reshape(1, D))


def kernel(x, edge_index, Ws, bs, gammas, betas):
    N, D = x.shape
    E = edge_index.shape[1]
    L = Ws.shape[0]
    EPWr = -(-E // NW)
    r0 = EPWr * NW - E
    src0 = jnp.concatenate(
        [edge_index[0], (jnp.arange(r0) % N).astype(jnp.int32)])
    dst0 = jnp.concatenate(
        [edge_index[1], N + (jnp.arange(r0) % 64).astype(jnp.int32)])
    EPW = -(-EPWr // K) * K
    NCH = EPW // K
    padw = EPW - EPWr
    padsrc = jnp.broadcast_to((jnp.arange(padw) % N).astype(jnp.int32),
                              (NW, padw))
    paddst = jnp.broadcast_to(
        N + (jnp.arange(padw) % 64).astype(jnp.int32), (NW, padw))
    srcp = jnp.concatenate([src0.reshape(NW, EPWr), padsrc], axis=1)
    dstp = jnp.concatenate([dst0.reshape(NW, EPWr), paddst], axis=1)
    ei = jnp.stack([srcp.reshape(NW, NCH, K), dstp.reshape(NW, NCH, K)],
                   axis=2)

    degp = _deg_kernel(N, NCH)(ei.reshape(NW, 2 * NCH, K))
    u = _tc_mm0(x, Ws[0])
    dinv, z = _tc_init(u, degp)

    agg = _agg_kernel(N, NCH, D)
    h = None
    for i in range(L - 1):
        S = agg(z, ei)
        h, z = _tc_layer(S, z, dinv, h, bs[i], gammas[i], betas[i],
                         Ws[i + 1], with_res=(i > 0))
    S = agg(z, ei)
    return _tc_final(S, z, dinv, bs[L - 1])


# speedup vs baseline: 1.0032x; 1.0032x over previous
import functools

import jax
import jax.numpy as jnp
from jax import lax
from jax.experimental import pallas as pl
from jax.experimental.pallas import tpu as pltpu
from jax.experimental.pallas import tpu_sc as plsc

NC = 2
NS = 16
NW = NC * NS
K = 128


CHR = 80


@functools.lru_cache(maxsize=None)
def _deg_kernel(N, NCH):
    NCHR = N // CHR
    MAXC = (NCHR + NS - 1) // NS
    mesh = plsc.VectorSubcoreMesh(core_axis_name="c", subcore_axis_name="s")

    @functools.partial(
        pl.kernel,
        out_type=jax.ShapeDtypeStruct((NC, N, 16), jnp.float32),
        mesh=mesh,
        scratch_types=[
            pltpu.VMEM_SHARED((N + 64, 16), jnp.float32),
            pltpu.VMEM((K,), jnp.int32),
            pltpu.VMEM((K,), jnp.int32),
            pltpu.VMEM((K, 16), jnp.float32),
            pltpu.VMEM((CHR, 16), jnp.float32),
            pltpu.SemaphoreType.DMA,
            pltpu.SemaphoreType.DMA,
        ],
    )
    def deg(ei_hbm, out_hbm, acc, idx0, idx1, ones_v, zbuf, isem0, isem1):
        c = lax.axis_index("c")
        s = lax.axis_index("s")
        wid = s * NC + c

        def fill(i, _):
            ones_v[i] = jnp.ones((16,), jnp.float32)
            zbuf[i % CHR] = jnp.zeros((16,), jnp.float32)
            return 0
        lax.fori_loop(0, K, fill, 0)

        def zchunk(k, _):
            idx = s + k * NS

            @pl.when(idx < NCHR)
            def _():
                pltpu.sync_copy(zbuf, acc.at[pl.ds(idx * CHR, CHR)])
            return 0
        lax.fori_loop(0, MAXC, zchunk, 0)
        plsc.subcore_barrier()

        def chunk(i, _):
            pltpu.sync_copy(ei_hbm.at[wid, 2 * i + 1], idx0)
            pltpu.sync_copy(ones_v, acc.at[idx0], add=True)
            return 0
        lax.fori_loop(0, NCH, chunk, 0)
        plsc.subcore_barrier()

        def rchunk(k, _):
            idx = s + k * NS

            @pl.when(idx < NCHR)
            def _():
                r0 = idx * CHR
                pltpu.sync_copy(acc.at[pl.ds(r0, CHR)], zbuf)
                pltpu.sync_copy(zbuf, out_hbm.at[c, pl.ds(r0, CHR)])
            return 0
        lax.fori_loop(0, MAXC, rchunk, 0)

    return deg


@functools.lru_cache(maxsize=None)
def _agg_kernel(N, NCH, D):
    NCHR = N // CHR
    MAXC = (NCHR + NS - 1) // NS
    mesh = plsc.VectorSubcoreMesh(core_axis_name="c", subcore_axis_name="s")

    @functools.partial(
        pl.kernel,
        out_type=jax.ShapeDtypeStruct((NC, N, D), jnp.float32),
        mesh=mesh,
        scratch_types=[
            pltpu.VMEM_SHARED((N + 64, D), jnp.float32),
            pltpu.VMEM((K,), jnp.int32),
            pltpu.VMEM((K,), jnp.int32),
            pltpu.VMEM((K,), jnp.int32),
            pltpu.VMEM((K,), jnp.int32),
            pltpu.VMEM((K, D), jnp.float32),
            pltpu.VMEM((K, D), jnp.float32),
            pltpu.SemaphoreType.DMA,
            pltpu.SemaphoreType.DMA,
            pltpu.SemaphoreType.DMA,
            pltpu.SemaphoreType.DMA,
        ],
    )
    def agg(z_hbm, ei_hbm, out_hbm, acc, srci0, srci1, dsti0, dsti1,
            rows0, rows1, isem0, isem1, rsem0, rsem1):
        c = lax.axis_index("c")
        s = lax.axis_index("s")
        wid = s * NC + c

        srcb = (srci0, srci1)
        dstb = (dsti0, dsti1)
        rowsb = (rows0, rows1)
        isems = (isem0, isem1)
        rsems = (rsem0, rsem1)

        pltpu.sync_copy(ei_hbm.at[wid, 0, 0], srci0)
        pltpu.sync_copy(ei_hbm.at[wid, 0, 1], dsti0)
        pltpu.async_copy(z_hbm.at[srci0], rows0, rsem0)
        pltpu.async_copy(ei_hbm.at[wid, 1, 0], srci1, isem1)
        pltpu.async_copy(ei_hbm.at[wid, 1, 1], dsti1, isem1)

        zbuf = rows1.at[pl.ds(0, CHR)]

        def fill(i, _):
            for j in range(D // 16):
                rows1[i, pl.ds(j * 16, 16)] = jnp.zeros((16,), jnp.float32)
            return 0
        lax.fori_loop(0, CHR, fill, 0)

        def zchunk(k, _):
            idx = s + k * NS

            @pl.when(idx < NCHR)
            def _():
                pltpu.async_copy(zbuf, acc.at[pl.ds(idx * CHR, CHR)],
                                 isem0)
            return 0
        lax.fori_loop(0, MAXC, zchunk, 0)

        def zwait(k, _):
            idx = s + k * NS

            @pl.when(idx < NCHR)
            def _():
                pltpu.make_async_copy(
                    zbuf, acc.at[pl.ds(idx * CHR, CHR)], isem0).wait()
            return 0
        lax.fori_loop(0, MAXC, zwait, 0)
        plsc.subcore_barrier()

        def step(cur, b):
            nb = 1 - b
            nxt = cur + 1

            @pl.when(nxt < NCH)
            def _():
                pltpu.make_async_copy(ei_hbm.at[wid, nxt, 0], srcb[nb],
                                      isems[nb]).wait()
                pltpu.make_async_copy(ei_hbm.at[wid, nxt, 1], dstb[nb],
                                      isems[nb]).wait()
                pltpu.async_copy(z_hbm.at[srcb[nb]], rowsb[nb], rsems[nb])
            pltpu.make_async_copy(z_hbm.at[srcb[b]], rowsb[b],
                                  rsems[b]).wait()
            pltpu.sync_copy(rowsb[b], acc.at[dstb[b]], add=True)
            nxt2 = cur + 2

            @pl.when(nxt2 < NCH)
            def _():
                pltpu.async_copy(ei_hbm.at[wid, nxt2, 0], srcb[b], isems[b])
                pltpu.async_copy(ei_hbm.at[wid, nxt2, 1], dstb[b], isems[b])

        def pair(i, _):
            for b in range(2):
                cur = 2 * i + b

                @pl.when(cur < NCH)
                def _():
                    step(cur, b)
            return 0
        lax.fori_loop(0, (NCH + 1) // 2, pair, 0)
        plsc.subcore_barrier()

        rb = (rows0.at[pl.ds(0, CHR)], rows1.at[pl.ds(0, CHR)])

        def rpair(i, _):
            for b in range(2):
                k = 2 * i + b
                idx = s + k * NS

                @pl.when(idx < NCHR)
                def _():
                    @pl.when(k >= 2)
                    def _():
                        prv = (s + (k - 2) * NS) * CHR
                        pltpu.make_async_copy(
                            rb[b], out_hbm.at[c, pl.ds(prv, CHR)],
                            rsems[b]).wait()
                    r0 = idx * CHR
                    pltpu.sync_copy(acc.at[pl.ds(r0, CHR)], rb[b])
                    pltpu.async_copy(rb[b], out_hbm.at[c, pl.ds(r0, CHR)],
                                     rsems[b])
            return 0
        lax.fori_loop(0, (MAXC + 1) // 2, rpair, 0)
        for b in range(2):
            pltpu.make_async_copy(rb[b], out_hbm.at[c, pl.ds(0, CHR)],
                                  rsems[b]).wait()

    return agg


def _tc_init(x, w0, degp):
    N, D = x.shape

    def body(x_ref, w_ref, degp_ref, dinv_ref, z_ref):
        deg = degp_ref[0, :, 0:1] + degp_ref[1, :, 0:1] + 1.0
        dinv = lax.rsqrt(deg)
        dinv_ref[...] = dinv
        z_ref[...] = jnp.dot(x_ref[...], w_ref[...],
                             preferred_element_type=jnp.float32) * dinv

    return pl.pallas_call(
        body,
        out_shape=[jax.ShapeDtypeStruct((N, 1), jnp.float32),
                   jax.ShapeDtypeStruct((N, D), jnp.float32)],
    )(x, w0, degp)


def _tc_layer(S, z, dinv, h_prev, b, g, be, w_next, with_res):
    N, D = z.shape

    def body(*refs):
        if with_res:
            (s_ref, z_ref, dinv_ref, h_ref, b_ref, g_ref, be_ref, w_ref,
             hn_ref, zn_ref) = refs
        else:
            (s_ref, z_ref, dinv_ref, b_ref, g_ref, be_ref, w_ref,
             hn_ref, zn_ref) = refs
        dinv = dinv_ref[...]
        agg = dinv * (s_ref[0] + s_ref[1] + z_ref[...]) + b_ref[...]
        mu = jnp.mean(agg, axis=0, keepdims=True)
        var = jnp.mean((agg - mu) ** 2, axis=0, keepdims=True)
        h = (agg - mu) * lax.rsqrt(var + 1e-5) * g_ref[...] + be_ref[...]
        h = jnp.maximum(h, 0.0)
        if with_res:
            h = h + h_ref[...]
        hn_ref[...] = h
        zn_ref[...] = jnp.dot(h, w_ref[...],
                              preferred_element_type=jnp.float32) * dinv

    args = [S, z, dinv] + ([h_prev] if with_res else []) + \
        [b.reshape(1, D), g.reshape(1, D), be.reshape(1, D), w_next]
    return pl.pallas_call(
        body,
        out_shape=[jax.ShapeDtypeStruct((N, D), jnp.float32),
                   jax.ShapeDtypeStruct((N, D), jnp.float32)],
    )(*args)


def _tc_final(S, z, dinv, b):
    N, D = z.shape

    def body(s_ref, z_ref, dinv_ref, b_ref, o_ref):
        o_ref[...] = dinv_ref[...] * (s_ref[0] + s_ref[1] + z_ref[...]) \
            + b_ref[...]

    return pl.pallas_call(
        body,
        out_shape=jax.ShapeDtypeStruct((N, D), jnp.float32),
    )(S, z, dinv, b.reshape(1, D))


def kernel(x, edge_index, Ws, bs, gammas, betas):
    N, D = x.shape
    E = edge_index.shape[1]
    L = Ws.shape[0]
    EPWr = -(-E // NW)
    r0 = EPWr * NW - E
    src0 = jnp.concatenate(
        [edge_index[0], (jnp.arange(r0) % N).astype(jnp.int32)])
    dst0 = jnp.concatenate(
        [edge_index[1], N + (jnp.arange(r0) % 64).astype(jnp.int32)])
    EPW = -(-EPWr // K) * K
    NCH = EPW // K
    padw = EPW - EPWr
    padsrc = jnp.broadcast_to((jnp.arange(padw) % N).astype(jnp.int32),
                              (NW, padw))
    paddst = jnp.broadcast_to(
        N + (jnp.arange(padw) % 64).astype(jnp.int32), (NW, padw))
    srcp = jnp.concatenate([src0.reshape(NW, EPWr), padsrc], axis=1)
    dstp = jnp.concatenate([dst0.reshape(NW, EPWr), paddst], axis=1)
    ei = jnp.stack([srcp.reshape(NW, NCH, K), dstp.reshape(NW, NCH, K)],
                   axis=2)

    degp = _deg_kernel(N, NCH)(ei.reshape(NW, 2 * NCH, K))
    dinv, z = _tc_init(x, Ws[0], degp)

    agg = _agg_kernel(N, NCH, D)
    h = None
    for i in range(L - 1):
        S = agg(z, ei)
        h, z = _tc_layer(S, z, dinv, h, bs[i], gammas[i], betas[i],
                         Ws[i + 1], with_res=(i > 0))
    S = agg(z, ei)
    return _tc_final(S, z, dinv, bs[L - 1])
